# baseline (speedup 1.0000x reference)
import functools

import numpy as np
import jax
import jax.numpy as jnp
from jax import lax
from jax.experimental import pallas as pl
from jax.experimental.pallas import tpu as pltpu
from jax.experimental.pallas import tpu_sc as plsc

_NUM_CLASSES = 100000
_B = 16384
_D = 128
_NPAIR = _NUM_CLASSES // 2
_BIA = 0.4

_NW = 32
_BPW = _B // _NW
_CHUNK = 128
_PPW = 1664
_PAIR_PAD = _NW * _PPW

np.random.seed(0)
_SHUFFLE = np.random.permutation(_NUM_CLASSES)
_IA = np.zeros((_PAIR_PAD,), np.int32)
_IB = np.zeros((_PAIR_PAD,), np.int32)
_IA[:_NPAIR] = _SHUFFLE[:_NPAIR]
_IB[:_NPAIR] = _SHUFFLE[_NPAIR:]


def _sc_body(label_hbm, feat_hbm, ctr_hbm, ia_hbm, ib_hbm,
             sq_out, part_out,
             idx_v, rows_v, feat_v, ia_v, ib_v, a_v, b_v, sq_v, acc_v, sem):
    c = lax.axis_index("c")
    s = lax.axis_index("s")
    wid = s * 2 + c

    accs = tuple(jnp.zeros((16,), jnp.float32) for _ in range(8))
    base = pl.multiple_of(wid * _BPW, _BPW)
    for ch in range(_BPW // _CHUNK):
        off = pl.multiple_of(base + ch * _CHUNK, _CHUNK)
        pltpu.sync_copy(label_hbm.at[pl.ds(off, _CHUNK)], idx_v)
        gather = pltpu.async_copy(ctr_hbm.at[idx_v], rows_v, sem)
        pltpu.sync_copy(feat_hbm.at[pl.ds(off, _CHUNK)], feat_v)
        gather.wait()

        def row_body(r, accs):
            out = []
            for v in range(8):
                f = feat_v[r, pl.ds(v * 16, 16)]
                cv = rows_v[r, pl.ds(v * 16, 16)]
                d = f - cv
                out.append(accs[v] + d * d)
            return tuple(out)

        accs = lax.fori_loop(0, _CHUNK, row_body, accs)
    part = accs[0] * jnp.float32(0.01)
    for v in range(1, 8):
        part = part + accs[v]
    acc_v[...] = part
    pltpu.sync_copy(acc_v, part_out.at[wid])

    pbase = pl.multiple_of(wid * _PPW, _CHUNK)
    for ch in range(_PPW // _CHUNK):
        off = pl.multiple_of(pbase + ch * _CHUNK, _CHUNK)
        pltpu.sync_copy(ia_hbm.at[pl.ds(off, _CHUNK)], ia_v)
        pltpu.sync_copy(ib_hbm.at[pl.ds(off, _CHUNK)], ib_v)
        cpa = pltpu.async_copy(ctr_hbm.at[ia_v], a_v, sem)
        cpb = pltpu.async_copy(ctr_hbm.at[ib_v], b_v, sem)
        cpa.wait()
        cpb.wait()

        def pair_body(p, carry):
            d0 = a_v[p, pl.ds(0, 16)] - b_v[p, pl.ds(0, 16)]
            d1 = a_v[p, pl.ds(16, 16)] - b_v[p, pl.ds(16, 16)]
            sq_v[p, :] = d0 * d0 + d1 * d1
            return carry

        lax.fori_loop(0, _CHUNK, pair_body, 0)
        pltpu.sync_copy(sq_v, sq_out.at[pl.ds(off, _CHUNK)])


def _tc_finish(sq_ref, part_ref, loss_ref, dloss_ref, dist_ref):
    sq = jnp.sum(sq_ref[...], axis=1)
    dist = jnp.sum(jnp.sqrt(sq)) * jnp.float32(1.0 / _NPAIR)
    loss = jnp.sum(part_ref[...]) * jnp.float32(0.5 / _B)
    loss_ref[...] = loss.reshape(1, 1)
    dloss_ref[...] = (jnp.float32(1.0) / (dist + jnp.float32(_BIA))).reshape(1, 1)
    dist_ref[...] = dist.reshape(1, 1)


def kernel(label, feat, centers):
    label = label.astype(jnp.int32)
    feat = feat.astype(jnp.float32)
    centers = centers.astype(jnp.float32)
    ia = jnp.asarray(_IA)
    ib = jnp.asarray(_IB)

    mesh = plsc.VectorSubcoreMesh(core_axis_name="c", subcore_axis_name="s")
    sc = pl.kernel(
        _sc_body,
        mesh=mesh,
        out_type=(
            jax.ShapeDtypeStruct((_PAIR_PAD, 16), jnp.float32),
            jax.ShapeDtypeStruct((_NW, 16), jnp.float32),
        ),
        scratch_types=[
            pltpu.VMEM((_CHUNK,), jnp.int32),
            pltpu.VMEM((_CHUNK, _D), jnp.float32),
            pltpu.VMEM((_CHUNK, _D), jnp.float32),
            pltpu.VMEM((_CHUNK,), jnp.int32),
            pltpu.VMEM((_CHUNK,), jnp.int32),
            pltpu.VMEM((_CHUNK, _D), jnp.float32),
            pltpu.VMEM((_CHUNK, _D), jnp.float32),
            pltpu.VMEM((_CHUNK, 16), jnp.float32),
            pltpu.VMEM((16,), jnp.float32),
            pltpu.SemaphoreType.DMA,
        ],
    )
    sq, parts = sc(label, feat, centers, ia, ib)

    loss2, dloss2, dist2 = pl.pallas_call(
        _tc_finish,
        out_shape=(
            jax.ShapeDtypeStruct((1, 1), jnp.float32),
            jax.ShapeDtypeStruct((1, 1), jnp.float32),
            jax.ShapeDtypeStruct((1, 1), jnp.float32),
        ),
    )(sq, parts.reshape(4, 128))

    return (loss2[0, 0], dloss2[0, 0], dist2[0, 0])


# speedup vs baseline: 1.0651x; 1.0651x over previous
import numpy as np
import jax
import jax.numpy as jnp
from jax import lax
from jax.experimental import pallas as pl
from jax.experimental.pallas import tpu as pltpu
from jax.experimental.pallas import tpu_sc as plsc

_NUM_CLASSES = 100000
_B = 16384
_D = 128
_NPAIR = _NUM_CLASSES // 2
_BIA = 0.4

_NW = 32
_BPW = _B // _NW
_CHUNK = 128
_LCH = _BPW // _CHUNK
_PCH = 13
_PPW = _PCH * _CHUNK
_PAIR_PAD = _NW * _PPW

np.random.seed(0)
_SHUFFLE = np.random.permutation(_NUM_CLASSES)
_IA = np.zeros((_PAIR_PAD,), np.int32)
_IB = np.zeros((_PAIR_PAD,), np.int32)
_IA[:_NPAIR] = _SHUFFLE[:_NPAIR]
_IB[:_NPAIR] = _SHUFFLE[_NPAIR:]


def _sc_body(label_hbm, feat_hbm, ctr_hbm, ia_hbm, ib_hbm,
             sq_out, part_out,
             lab_v, ia_v, ib_v, rows0, rows1, feat0, feat1, sq0, sq1, acc_v,
             sem0, sem1, semo):
    c = lax.axis_index("c")
    s = lax.axis_index("s")
    wid = s * 2 + c
    lbase = pl.multiple_of(wid * _BPW, _BPW)
    pbase = pl.multiple_of(wid * _PPW, _CHUNK)

    rows = (rows0, rows1)
    feats = (feat0, feat1)
    sqs = (sq0, sq1)
    sems = (sem0, sem1)

    cps = (pltpu.async_copy(label_hbm.at[pl.ds(lbase, _BPW)], lab_v, sem0),
           pltpu.async_copy(ia_hbm.at[pl.ds(pbase, _PPW)], ia_v, sem0),
           pltpu.async_copy(ib_hbm.at[pl.ds(pbase, _PPW)], ib_v, sem0))
    for cp in cps:
        cp.wait()

    def issue(item, buf):
        if item < _LCH:
            off = pl.multiple_of(lbase + item * _CHUNK, _CHUNK)
            return (
                pltpu.async_copy(
                    ctr_hbm.at[lab_v.at[pl.ds(item * _CHUNK, _CHUNK)]],
                    rows[buf], sems[buf]),
                pltpu.async_copy(feat_hbm.at[pl.ds(off, _CHUNK)],
                                 feats[buf], sems[buf]),
            )
        ch = item - _LCH
        return (
            pltpu.async_copy(ctr_hbm.at[ia_v.at[pl.ds(ch * _CHUNK, _CHUNK)]],
                             rows[buf], sems[buf]),
            pltpu.async_copy(ctr_hbm.at[ib_v.at[pl.ds(ch * _CHUNK, _CHUNK)]],
                             feats[buf], sems[buf]),
        )

    n_items = _LCH + _PCH
    accs = tuple(jnp.zeros((16,), jnp.float32) for _ in range(8))
    pending = [None, None]
    out_pending = [None, None]
    pending[0] = issue(0, 0)

    for item in range(n_items):
        buf = item & 1
        if item + 1 < n_items:
            pending[1 - buf] = issue(item + 1, 1 - buf)
        for cp in pending[buf]:
            cp.wait()

        if item < _LCH:
            fv, rv = feats[buf], rows[buf]

            def row_body(i, accs, fv=fv, rv=rv):
                r = i * 2
                out = list(accs)
                for k in range(2):
                    for v in range(8):
                        d = fv[r + k, pl.ds(v * 16, 16)] - rv[r + k, pl.ds(v * 16, 16)]
                        out[v] = out[v] + d * d
                return tuple(out)

            accs = lax.fori_loop(0, _CHUNK // 2, row_body, accs)
        else:
            ch = item - _LCH
            av, bv, sv = rows[buf], feats[buf], sqs[buf]
            if out_pending[buf] is not None:
                out_pending[buf].wait()

            def pair_body(i, carry, av=av, bv=bv, sv=sv):
                p = i * 4
                for k in range(4):
                    d0 = av[p + k, pl.ds(0, 16)] - bv[p + k, pl.ds(0, 16)]
                    d1 = av[p + k, pl.ds(16, 16)] - bv[p + k, pl.ds(16, 16)]
                    sv[p + k, :] = d0 * d0 + d1 * d1
                return carry

            lax.fori_loop(0, _CHUNK // 4, pair_body, 0)
            off = pl.multiple_of(pbase + ch * _CHUNK, _CHUNK)
            out_pending[buf] = pltpu.async_copy(
                sqs[buf], sq_out.at[pl.ds(off, _CHUNK)], semo)

    part = accs[0] * jnp.float32(0.01)
    for v in range(1, 8):
        part = part + accs[v]
    acc_v[...] = part
    pltpu.sync_copy(acc_v, part_out.at[wid])
    for cp in out_pending:
        if cp is not None:
            cp.wait()


def _tc_finish(sq_ref, part_ref, loss_ref, dloss_ref, dist_ref):
    sq = jnp.sum(sq_ref[...], axis=1)
    dist = jnp.sum(jnp.sqrt(sq)) * jnp.float32(1.0 / _NPAIR)
    loss = jnp.sum(part_ref[...]) * jnp.float32(0.5 / _B)
    loss_ref[...] = loss.reshape(1, 1)
    dloss_ref[...] = (jnp.float32(1.0) / (dist + jnp.float32(_BIA))).reshape(1, 1)
    dist_ref[...] = dist.reshape(1, 1)


def kernel(label, feat, centers):
    label = label.astype(jnp.int32)
    feat = feat.astype(jnp.float32)
    centers = centers.astype(jnp.float32)
    ia = jnp.asarray(_IA)
    ib = jnp.asarray(_IB)

    mesh = plsc.VectorSubcoreMesh(core_axis_name="c", subcore_axis_name="s")
    sc = pl.kernel(
        _sc_body,
        mesh=mesh,
        out_type=(
            jax.ShapeDtypeStruct((_PAIR_PAD, 16), jnp.float32),
            jax.ShapeDtypeStruct((_NW, 16), jnp.float32),
        ),
        scratch_types=[
            pltpu.VMEM((_BPW,), jnp.int32),
            pltpu.VMEM((_PPW,), jnp.int32),
            pltpu.VMEM((_PPW,), jnp.int32),
            pltpu.VMEM((_CHUNK, _D), jnp.float32),
            pltpu.VMEM((_CHUNK, _D), jnp.float32),
            pltpu.VMEM((_CHUNK, _D), jnp.float32),
            pltpu.VMEM((_CHUNK, _D), jnp.float32),
            pltpu.VMEM((_CHUNK, 16), jnp.float32),
            pltpu.VMEM((_CHUNK, 16), jnp.float32),
            pltpu.VMEM((16,), jnp.float32),
            pltpu.SemaphoreType.DMA,
            pltpu.SemaphoreType.DMA,
            pltpu.SemaphoreType.DMA,
        ],
    )
    sq, parts = sc(label, feat, centers, ia, ib)

    loss2, dloss2, dist2 = pl.pallas_call(
        _tc_finish,
        out_shape=(
            jax.ShapeDtypeStruct((1, 1), jnp.float32),
            jax.ShapeDtypeStruct((1, 1), jnp.float32),
            jax.ShapeDtypeStruct((1, 1), jnp.float32),
        ),
    )(sq, parts.reshape(4, 128))

    return (loss2[0, 0], dloss2[0, 0], dist2[0, 0])
